# baseline (speedup 1.0000x reference)
import jax
import jax.numpy as jnp
from jax import lax
from jax.experimental import pallas as pl
from jax.experimental.pallas import tpu as pltpu

_R = 2048
_C = 128
_P = _R * _C
_LOG2P = 18


def _rowmajor_cumsum(L):
    col = lax.broadcasted_iota(jnp.int32, (_R, _C), 1)
    for s in (1, 2, 4, 8, 16, 32, 64):
        L = L + jnp.where(col >= s, jnp.roll(L, s, axis=1), 0)
    rowtot = L[:, _C - 1:_C]
    row = lax.broadcasted_iota(jnp.int32, (_R, 1), 0)
    R = rowtot
    for s in (1, 2, 4, 8, 16, 32, 64, 128, 256, 512, 1024):
        R = R + jnp.where(row >= s, jnp.roll(R, s, axis=0), 0)
    return L + (R - rowtot)


def _pack_key(d, t):
    sign = (2 * t - 1).astype(jnp.float32)
    e = 1.0 - d * sign
    b = lax.bitcast_convert_type(e, jnp.int32)
    asc = jnp.where(b >= 0, b, b ^ jnp.int32(0x7FFFFFFF))
    return ((asc ^ jnp.int32(-1)) & jnp.int32(-2)) | t


def _loss_body(lp_ref, ln_ref, tt_ref, out_ref):
    keys = [_pack_key(lp_ref[i] - ln_ref[i], tt_ref[i]) for i in range(2)]

    idx = (lax.broadcasted_iota(jnp.int32, (_R, _C), 0) * _C
           + lax.broadcasted_iota(jnp.int32, (_R, _C), 1))

    row1 = lax.broadcasted_iota(jnp.int32, (_R, 1), 0)
    col1 = lax.broadcasted_iota(jnp.int32, (1, _C), 1)

    def stage(kk, xs):
        kv = jnp.left_shift(jnp.int32(1), kk)
        mrow = (row1 & lax.shift_right_logical(kv, 7)) != 0
        mcol = (col1 & kv) != 0
        flip = jnp.where(jnp.logical_xor(mrow, mcol), jnp.int32(-1),
                         jnp.int32(0))
        xs = tuple(x ^ flip for x in xs)

        def row_pass(p, xxs):
            jrow = jnp.left_shift(jnp.int32(1), kk - 8 - p)
            cb = (row1 & jrow) != 0
            out = []
            for xx in xxs:
                down = pltpu.roll(xx, jrow, axis=0)
                up = pltpu.roll(xx, _R - jrow, axis=0)
                out.append(jnp.where(cb, jnp.maximum(xx, down),
                                     jnp.minimum(xx, up)))
            return tuple(out)

        def col_pass(p, xxs):
            jv = jnp.left_shift(jnp.int32(1), jnp.minimum(kk, 7) - 1 - p)
            cb = (col1 & jv) != 0
            out = []
            for xx in xxs:
                down = pltpu.roll(xx, jv, axis=1)
                up = pltpu.roll(xx, _C - jv, axis=1)
                out.append(jnp.where(cb, jnp.maximum(xx, down),
                                     jnp.minimum(xx, up)))
            return tuple(out)

        xs = lax.fori_loop(0, jnp.maximum(kk - 7, 0), row_pass, xs)
        xs = lax.fori_loop(0, jnp.minimum(kk, 7), col_pass, xs)
        return tuple(x ^ flip for x in xs)

    keys = lax.fori_loop(1, _LOG2P + 1, stage, tuple(keys))

    for i, key in enumerate(keys):
        lab = key & 1
        ascr = key ^ jnp.int32(-1)
        br = jnp.where(ascr >= 0, ascr, ascr ^ jnp.int32(0x7FFFFFFF))
        es = lax.bitcast_convert_type(br, jnp.float32)
        a = jnp.maximum(es, 0.0)

        colnext = jnp.roll(a, -1, axis=1)
        rownext0 = jnp.roll(a, -1, axis=0)[:, 0:1]
        lastcol = (idx & (_C - 1)) == (_C - 1)
        an = jnp.where(lastcol, rownext0, colnext)
        an = jnp.where(idx == _P - 1, 0.0, an)

        cp = _rowmajor_cumsum(lab).astype(jnp.float32)
        gt = cp[_R - 1:_R, _C - 1:_C]
        i_f = (idx + 1).astype(jnp.float32)
        jac = 1.0 - (gt - cp) / (gt + i_f - cp)

        out_ref[i, 0, :] = jnp.broadcast_to(jnp.sum((a - an) * jac), (_C,))


def kernel(logits, targets):
    B = logits.shape[0]
    lp = logits[:, 1].reshape(B, _R, _C)
    ln = logits[:, 0].reshape(B, _R, _C)
    tt = targets.astype(jnp.int32).reshape(B, _R, _C)
    out = pl.pallas_call(
        _loss_body,
        grid=(B // 2,),
        in_specs=[pl.BlockSpec((2, _R, _C), lambda b: (b, 0, 0))] * 3,
        out_specs=pl.BlockSpec((2, 1, _C), lambda b: (b, 0, 0)),
        out_shape=jax.ShapeDtypeStruct((B, 1, _C), jnp.float32),
        compiler_params=pltpu.CompilerParams(
            dimension_semantics=("arbitrary",)),
    )(lp, ln, tt)
    return jnp.mean(out[:, 0, 0])


# speedup vs baseline: 1.2566x; 1.2566x over previous
import jax
import jax.numpy as jnp
from jax import lax
from jax.experimental import pallas as pl
from jax.experimental.pallas import tpu as pltpu

_R = 2048
_C = 128
_P = _R * _C
_LOG2P = 18


def _rowmajor_cumsum(L):
    col = lax.broadcasted_iota(jnp.int32, (_R, _C), 1)
    for s in (1, 2, 4, 8, 16, 32, 64):
        L = L + jnp.where(col >= s, jnp.roll(L, s, axis=1), 0)
    rowtot = L[:, _C - 1:_C]
    row = lax.broadcasted_iota(jnp.int32, (_R, 1), 0)
    R = rowtot
    for s in (1, 2, 4, 8, 16, 32, 64, 128, 256, 512, 1024):
        R = R + jnp.where(row >= s, jnp.roll(R, s, axis=0), 0)
    return L + (R - rowtot)


def _pack_key(d, t):
    sign = (2 * t - 1).astype(jnp.float32)
    e = 1.0 - d * sign
    b = lax.bitcast_convert_type(e, jnp.int32)
    asc = jnp.where(b >= 0, b, b ^ jnp.int32(0x7FFFFFFF))
    return ((asc ^ jnp.int32(-1)) & jnp.int32(-2)) | t


def _loss_body(lp_ref, ln_ref, tt_ref, out_ref):
    key = _pack_key(lp_ref[0] - ln_ref[0], tt_ref[0])

    idx = (lax.broadcasted_iota(jnp.int32, (_R, _C), 0) * _C
           + lax.broadcasted_iota(jnp.int32, (_R, _C), 1))

    row1 = lax.broadcasted_iota(jnp.int32, (_R, 1), 0)
    col1 = lax.broadcasted_iota(jnp.int32, (1, _C), 1)
    col_cbs = [(col1 & (1 << t)) != 0 for t in range(7)]

    def lane_suffix(xx):
        for t in (6, 5, 4, 3, 2, 1, 0):
            dist = 1 << t
            down = pltpu.roll(xx, dist, axis=1)
            up = pltpu.roll(xx, _C - dist, axis=1)
            xx = jnp.where(col_cbs[t], jnp.maximum(xx, down),
                           jnp.minimum(xx, up))
        return xx

    def flip_mask(kk):
        kv = jnp.left_shift(jnp.int32(1), kk)
        mrow = (row1 & lax.shift_right_logical(kv, 7)) != 0
        mcol = (col1 & kv) != 0
        return jnp.where(jnp.logical_xor(mrow, mcol), jnp.int32(-1),
                         jnp.int32(0))

    def small_stage(kk, x):
        flip = flip_mask(kk)
        x = x ^ flip

        def col_pass(p, xx):
            jv = jnp.left_shift(jnp.int32(1), kk - 1 - p)
            down = pltpu.roll(xx, jv, axis=1)
            up = pltpu.roll(xx, _C - jv, axis=1)
            return jnp.where((col1 & jv) != 0, jnp.maximum(xx, down),
                             jnp.minimum(xx, up))

        x = lax.fori_loop(0, kk, col_pass, x)
        return x ^ flip

    def big_stage(kk, x):
        flip = flip_mask(kk)
        x = x ^ flip

        def row_pass(p, xx):
            jrow = jnp.left_shift(jnp.int32(1), kk - 8 - p)
            down = pltpu.roll(xx, jrow, axis=0)
            up = pltpu.roll(xx, _R - jrow, axis=0)
            return jnp.where((row1 & jrow) != 0, jnp.maximum(xx, down),
                             jnp.minimum(xx, up))

        x = lax.fori_loop(0, kk - 7, row_pass, x)
        x = lane_suffix(x)
        return x ^ flip

    key = lax.fori_loop(1, 7, small_stage, key)
    key = lax.fori_loop(7, _LOG2P + 1, big_stage, key)

    lab = key & 1
    ascr = key ^ jnp.int32(-1)
    br = jnp.where(ascr >= 0, ascr, ascr ^ jnp.int32(0x7FFFFFFF))
    es = lax.bitcast_convert_type(br, jnp.float32)
    a = jnp.maximum(es, 0.0)

    colnext = jnp.roll(a, -1, axis=1)
    rownext0 = jnp.roll(a, -1, axis=0)[:, 0:1]
    lastcol = (idx & (_C - 1)) == (_C - 1)
    an = jnp.where(lastcol, rownext0, colnext)
    an = jnp.where(idx == _P - 1, 0.0, an)

    cp = _rowmajor_cumsum(lab).astype(jnp.float32)
    gt = cp[_R - 1:_R, _C - 1:_C]
    i_f = (idx + 1).astype(jnp.float32)
    jac = 1.0 - (gt - cp) / (gt + i_f - cp)

    out_ref[0, 0, :] = jnp.broadcast_to(jnp.sum((a - an) * jac), (_C,))


def kernel(logits, targets):
    B = logits.shape[0]
    lp = logits[:, 1].reshape(B, _R, _C)
    ln = logits[:, 0].reshape(B, _R, _C)
    tt = targets.astype(jnp.int32).reshape(B, _R, _C)
    out = pl.pallas_call(
        _loss_body,
        grid=(B,),
        in_specs=[pl.BlockSpec((1, _R, _C), lambda b: (b, 0, 0))] * 3,
        out_specs=pl.BlockSpec((1, 1, _C), lambda b: (b, 0, 0)),
        out_shape=jax.ShapeDtypeStruct((B, 1, _C), jnp.float32),
        compiler_params=pltpu.CompilerParams(
            dimension_semantics=("arbitrary",)),
    )(lp, ln, tt)
    return jnp.mean(out[:, 0, 0])
